# Initial kernel scaffold; baseline (speedup 1.0000x reference)
import jax
import jax.numpy as jnp
from jax.experimental import pallas as pl


def kernel(x, table):
    raise NotImplementedError("write your pallas kernel here")


# speedup vs baseline: 1.5758x; 1.5758x over previous
import jax
import jax.numpy as jnp
from jax import lax
from jax.experimental import pallas as pl
from jax.experimental.pallas import tpu as pltpu
from jax.experimental.pallas import tpu_sc as plsc

_NUM_WORKERS = 32
_CHUNK = 128


def kernel(x, table):
    batch, seq = x.shape
    _, emb = table.shape
    n = batch * seq
    idx = x.reshape(n)
    per_worker = n // _NUM_WORKERS
    num_chunks = per_worker // _CHUNK

    mesh = plsc.VectorSubcoreMesh(core_axis_name="c", subcore_axis_name="s")

    @pl.kernel(
        out_type=jax.ShapeDtypeStruct((n, emb), table.dtype),
        mesh=mesh,
        compiler_params=pltpu.CompilerParams(use_tc_tiling_on_sc=False),
        scratch_types=[
            pltpu.VMEM((_CHUNK,), jnp.int32),
            pltpu.VMEM((_CHUNK, emb), table.dtype),
            pltpu.SemaphoreType.DMA,
        ],
    )
    def gather_kernel(table_hbm, idx_hbm, out_hbm, idx_v, rows_v, sem):
        wid = lax.axis_index("s") * 2 + lax.axis_index("c")
        base = wid * per_worker

        @pl.loop(0, num_chunks)
        def _(c):
            off = base + c * _CHUNK
            pltpu.sync_copy(idx_hbm.at[pl.ds(off, _CHUNK)], idx_v)
            pltpu.async_copy(table_hbm.at[idx_v], rows_v, sem).wait()
            pltpu.sync_copy(rows_v, out_hbm.at[pl.ds(off, _CHUNK)])

    out = gather_kernel(table, idx)
    return out.reshape(batch, seq, emb)


# speedup vs baseline: 1.8742x; 1.1894x over previous
import jax
import jax.numpy as jnp
from jax import lax
from jax.experimental import pallas as pl
from jax.experimental.pallas import tpu as pltpu
from jax.experimental.pallas import tpu_sc as plsc

_NUM_WORKERS = 32
_CHUNK = 128
_NBUF = 8
_LAG = 4


def kernel(x, table):
    batch, seq = x.shape
    _, emb = table.shape
    n = batch * seq
    idx = x.reshape(n)
    per_worker = n // _NUM_WORKERS
    num_chunks = per_worker // _CHUNK
    num_groups = num_chunks // _NBUF

    mesh = plsc.VectorSubcoreMesh(core_axis_name="c", subcore_axis_name="s")

    @pl.kernel(
        out_type=jax.ShapeDtypeStruct((n, emb), table.dtype),
        mesh=mesh,
        compiler_params=pltpu.CompilerParams(use_tc_tiling_on_sc=False),
        scratch_types=[
            pltpu.VMEM((per_worker,), jnp.int32),
            [pltpu.VMEM((_CHUNK, emb), table.dtype) for _ in range(_NBUF)],
            [pltpu.SemaphoreType.DMA for _ in range(_NBUF)],
            [pltpu.SemaphoreType.DMA for _ in range(_NBUF)],
        ],
    )
    def gather_kernel(table_hbm, idx_hbm, out_hbm, idx_all, rows, gsem, wsem):
        wid = lax.axis_index("s") * 2 + lax.axis_index("c")
        base = wid * per_worker
        pltpu.sync_copy(idx_hbm.at[pl.ds(base, per_worker)], idx_all)

        def start_gather(c, b):
            pltpu.async_copy(
                table_hbm.at[idx_all.at[pl.ds(c * _CHUNK, _CHUNK)]],
                rows[b],
                gsem[b],
            )

        def wait_gather(c, b):
            pltpu.make_async_copy(
                table_hbm.at[idx_all.at[pl.ds(c * _CHUNK, _CHUNK)]],
                rows[b],
                gsem[b],
            ).wait()

        def start_wb(c, b):
            pltpu.async_copy(
                rows[b],
                out_hbm.at[pl.ds(base + c * _CHUNK, _CHUNK)],
                wsem[b],
            )

        def wait_wb(c, b):
            pltpu.make_async_copy(
                rows[b],
                out_hbm.at[pl.ds(base + c * _CHUNK, _CHUNK)],
                wsem[b],
            ).wait()

        for i in range(_NBUF):
            start_gather(i, i)
            if i >= _LAG:
                d = i - _LAG
                wait_gather(d, d % _NBUF)
                start_wb(d, d % _NBUF)

        @pl.loop(1, num_groups)
        def _(k):
            c0 = k * _NBUF
            for i in range(_NBUF):
                c = c0 + i
                wait_wb(c - _NBUF, i)
                start_gather(c, i)
                d = c - _LAG
                bd = (i + _NBUF - _LAG) % _NBUF
                wait_gather(d, bd)
                start_wb(d, bd)

        for d in range(num_chunks - _LAG, num_chunks):
            wait_gather(d, d % _NBUF)
            start_wb(d, d % _NBUF)
        for b in range(_NBUF):
            wait_wb(num_chunks - _NBUF + b, b)

    out = gather_kernel(table, idx)
    return out.reshape(batch, seq, emb)
